# baseline (speedup 1.0000x reference)
import functools

import jax
import jax.numpy as jnp
import numpy as np
from jax import lax
from jax.experimental import pallas as pl
from jax.experimental.pallas import tpu as pltpu
from jax.experimental.pallas import tpu_sc as plsc

V = 100000
B = 1024
IGN_COL = V - 100
S = float(np.float32(0.1 / (V - 2)))
C = 0.9
LS = float(np.log(np.float32(S)))
LC = float(np.log(np.float32(C)))
CONST = B * ((V - 2) * S * LS + C * LC)


_BB = 8


def _dense_sum_body(x_ref, acc_ref):
    @pl.when(pl.program_id(0) == 0)
    def _():
        acc_ref[0, 0] = 0.0

    acc_ref[0, 0] += jnp.sum(x_ref[...])


def _dense_sum(output):
    return pl.pallas_call(
        _dense_sum_body,
        grid=(B // _BB,),
        in_specs=[pl.BlockSpec((_BB, V), lambda i: (i, 0))],
        out_specs=pl.BlockSpec(memory_space=pltpu.SMEM),
        out_shape=jax.ShapeDtypeStruct((1, 1), jnp.float32),
    )(output)


_INFO = plsc.get_sparse_core_info()
_NC, _NS, _L = _INFO.num_cores, _INFO.num_subcores, _INFO.num_lanes
_NW = _NC * _NS
_BPW = B // _NW
_NCH = _BPW // _L

_MESH = plsc.VectorSubcoreMesh(core_axis_name="c", subcore_axis_name="s")


@functools.partial(
    pl.kernel,
    mesh=_MESH,
    out_type=jax.ShapeDtypeStruct((_NW, _L), jnp.float32),
    scratch_types=[
        pltpu.VMEM((_BPW,), jnp.int32),
        pltpu.VMEM((2 * _BPW,), jnp.int32),
        pltpu.VMEM((2 * _BPW,), jnp.float32),
        pltpu.VMEM((_L,), jnp.float32),
        pltpu.SemaphoreType.DMA,
    ],
)
def _sc_gather(flat_hbm, tgt_hbm, out_hbm, tgt_v, idx_v, vals_v, row_v, sem):
    wid = lax.axis_index("s") * _NC + lax.axis_index("c")
    base = wid * _BPW
    pltpu.sync_copy(tgt_hbm.at[pl.ds(base, _BPW)], tgt_v)
    lane = lax.iota(jnp.int32, _L)
    for k in range(_NCH):
        rows = (base + k * _L + lane) * V
        idx_v[pl.ds(k * _L, _L)] = rows + tgt_v[pl.ds(k * _L, _L)]
        idx_v[pl.ds(_BPW + k * _L, _L)] = rows + IGN_COL
    pltpu.async_copy(flat_hbm.at[idx_v], vals_v, sem).wait()
    acc = jnp.zeros((_L,), jnp.float32)
    for k in range(_NCH):
        g = vals_v[pl.ds(k * _L, _L)]
        gcol = vals_v[pl.ds(_BPW + k * _L, _L)]
        is_ign = tgt_v[pl.ds(k * _L, _L)] == IGN_COL
        acc = acc + (S - C) * g + jnp.where(is_ign, S * LS, S * gcol)
    row_v[...] = acc
    pltpu.sync_copy(row_v, out_hbm.at[wid])


def kernel(output, target, one_hot):
    total = _dense_sum(output)[0, 0]
    return jnp.float32(CONST) - jnp.float32(S) * total


# speedup vs baseline: 3.3062x; 3.3062x over previous
import functools

import jax
import jax.numpy as jnp
import numpy as np
from jax import lax
from jax.experimental import pallas as pl
from jax.experimental.pallas import tpu as pltpu
from jax.experimental.pallas import tpu_sc as plsc

V = 100000
B = 1024
IGN_COL = V - 100
S = float(np.float32(0.1 / (V - 2)))
C = 0.9
LS = float(np.log(np.float32(S)))
LC = float(np.log(np.float32(C)))
CONST = B * ((V - 2) * S * LS + C * LC)


_VB = 2000


def _wsum_body(t_ref, x_ref, acc_ref):
    @pl.when(pl.program_id(0) == 0)
    def _():
        acc_ref[0, 0] = 0.0

    x = x_ref[...]
    t = t_ref[...]
    v = lax.broadcasted_iota(jnp.int32, (_VB, B), 0) + pl.program_id(0) * _VB
    w = jnp.where(v == t, S - C, 0.0) - S
    w = jnp.where((v == IGN_COL) & (t != IGN_COL), w + S, w)
    acc_ref[0, 0] += jnp.sum(x * w)


def _weighted_sum(xt, target):
    return pl.pallas_call(
        _wsum_body,
        grid=(V // _VB,),
        in_specs=[
            pl.BlockSpec((1, B), lambda i: (0, 0)),
            pl.BlockSpec((_VB, B), lambda i: (i, 0)),
        ],
        out_specs=pl.BlockSpec(memory_space=pltpu.SMEM),
        out_shape=jax.ShapeDtypeStruct((1, 1), jnp.float32),
    )(target.reshape(1, B), xt)


_INFO = plsc.get_sparse_core_info()
_NC, _NS, _L = _INFO.num_cores, _INFO.num_subcores, _INFO.num_lanes
_NW = _NC * _NS
_BPW = B // _NW
_NCH = _BPW // _L

_MESH = plsc.VectorSubcoreMesh(core_axis_name="c", subcore_axis_name="s")


@functools.partial(
    pl.kernel,
    mesh=_MESH,
    out_type=jax.ShapeDtypeStruct((_NW, _L), jnp.float32),
    scratch_types=[
        pltpu.VMEM((_BPW,), jnp.int32),
        pltpu.VMEM((_L,), jnp.float32),
    ],
)
def _sc_ign_term(tgt_hbm, out_hbm, tgt_v, row_v):
    wid = lax.axis_index("s") * _NC + lax.axis_index("c")
    base = wid * _BPW
    pltpu.sync_copy(tgt_hbm.at[pl.ds(base, _BPW)], tgt_v)
    acc = jnp.zeros((_L,), jnp.float32)
    for k in range(_NCH):
        is_ign = tgt_v[pl.ds(k * _L, _L)] == IGN_COL
        acc = acc + jnp.where(is_ign, S * LS, 0.0)
    row_v[...] = acc
    pltpu.sync_copy(row_v, out_hbm.at[wid])


def kernel(output, target, one_hot):
    wsum = _weighted_sum(output.T, target)[0, 0]
    parts = _sc_ign_term(target)
    return jnp.float32(CONST) + wsum + jnp.sum(parts)


# speedup vs baseline: 3.6194x; 1.0947x over previous
import functools

import jax
import jax.numpy as jnp
import numpy as np
from jax import lax
from jax.experimental import pallas as pl
from jax.experimental.pallas import tpu as pltpu
from jax.experimental.pallas import tpu_sc as plsc

V = 100000
B = 1024
IGN_COL = V - 100
S = float(np.float32(0.1 / (V - 2)))
C = 0.9
LS = float(np.log(np.float32(S)))
LC = float(np.log(np.float32(C)))
CONST = B * ((V - 2) * S * LS + C * LC)


_VB = 4000


def _wsum_body(t_ref, x_ref, acc_ref):
    @pl.when(pl.program_id(0) == 0)
    def _():
        acc_ref[0, 0] = 0.0

    x = x_ref[...]
    t = t_ref[...]
    v = lax.broadcasted_iota(jnp.int32, (_VB, B), 0) + pl.program_id(0) * _VB
    w = jnp.where(v == t, S - C, 0.0) - S
    w = jnp.where((v == IGN_COL) & (t != IGN_COL), w + S, w)
    acc_ref[0, 0] += jnp.sum(x * w)


def _weighted_sum(xt, target):
    return pl.pallas_call(
        _wsum_body,
        grid=(V // _VB,),
        in_specs=[
            pl.BlockSpec((1, B), lambda i: (0, 0)),
            pl.BlockSpec((_VB, B), lambda i: (i, 0)),
        ],
        out_specs=pl.BlockSpec(memory_space=pltpu.SMEM),
        out_shape=jax.ShapeDtypeStruct((1, 1), jnp.float32),
    )(target.reshape(1, B), xt)


_INFO = plsc.get_sparse_core_info()
_NC, _NS, _L = _INFO.num_cores, _INFO.num_subcores, _INFO.num_lanes
_NW = _NC * _NS
_BPW = B // _NW
_NCH = _BPW // _L

_MESH = plsc.VectorSubcoreMesh(core_axis_name="c", subcore_axis_name="s")


@functools.partial(
    pl.kernel,
    mesh=_MESH,
    out_type=jax.ShapeDtypeStruct((_NW, _L), jnp.float32),
    scratch_types=[
        pltpu.VMEM((_BPW,), jnp.int32),
        pltpu.VMEM((_L,), jnp.float32),
    ],
)
def _sc_ign_term(tgt_hbm, out_hbm, tgt_v, row_v):
    wid = lax.axis_index("s") * _NC + lax.axis_index("c")
    base = wid * _BPW
    pltpu.sync_copy(tgt_hbm.at[pl.ds(base, _BPW)], tgt_v)
    acc = jnp.zeros((_L,), jnp.float32)
    for k in range(_NCH):
        is_ign = tgt_v[pl.ds(k * _L, _L)] == IGN_COL
        acc = acc + jnp.where(is_ign, S * LS, 0.0)
    row_v[...] = acc
    pltpu.sync_copy(row_v, out_hbm.at[wid])


def kernel(output, target, one_hot):
    wsum = _weighted_sum(output.T, target)[0, 0]
    parts = _sc_ign_term(target)
    return jnp.float32(CONST) + wsum + jnp.sum(parts)


# speedup vs baseline: 3.6589x; 1.0109x over previous
import functools

import jax
import jax.numpy as jnp
import numpy as np
from jax import lax
from jax.experimental import pallas as pl
from jax.experimental.pallas import tpu as pltpu
from jax.experimental.pallas import tpu_sc as plsc

V = 100000
B = 1024
IGN_COL = V - 100
S = float(np.float32(0.1 / (V - 2)))
C = 0.9
LS = float(np.log(np.float32(S)))
LC = float(np.log(np.float32(C)))
CONST = B * ((V - 2) * S * LS + C * LC)


_VB = 5000


def _wsum_body(t_ref, x_ref, acc_ref):
    @pl.when(pl.program_id(0) == 0)
    def _():
        acc_ref[0, 0] = 0.0

    x = x_ref[...]
    t = t_ref[...]
    v = lax.broadcasted_iota(jnp.int32, (_VB, B), 0) + pl.program_id(0) * _VB
    w = jnp.where(v == t, S - C, 0.0) - S
    w = jnp.where((v == IGN_COL) & (t != IGN_COL), w + S, w)
    acc_ref[0, 0] += jnp.sum(x * w)


def _weighted_sum(xt, target):
    return pl.pallas_call(
        _wsum_body,
        grid=(V // _VB,),
        in_specs=[
            pl.BlockSpec((1, B), lambda i: (0, 0)),
            pl.BlockSpec((_VB, B), lambda i: (i, 0)),
        ],
        out_specs=pl.BlockSpec(memory_space=pltpu.SMEM),
        out_shape=jax.ShapeDtypeStruct((1, 1), jnp.float32),
    )(target.reshape(1, B), xt)


_INFO = plsc.get_sparse_core_info()
_NC, _NS, _L = _INFO.num_cores, _INFO.num_subcores, _INFO.num_lanes
_NW = _NC * _NS
_BPW = B // _NW
_NCH = _BPW // _L

_MESH = plsc.VectorSubcoreMesh(core_axis_name="c", subcore_axis_name="s")


@functools.partial(
    pl.kernel,
    mesh=_MESH,
    out_type=jax.ShapeDtypeStruct((_NW, _L), jnp.float32),
    scratch_types=[
        pltpu.VMEM((_BPW,), jnp.int32),
        pltpu.VMEM((_L,), jnp.float32),
    ],
)
def _sc_ign_term(tgt_hbm, out_hbm, tgt_v, row_v):
    wid = lax.axis_index("s") * _NC + lax.axis_index("c")
    base = wid * _BPW
    pltpu.sync_copy(tgt_hbm.at[pl.ds(base, _BPW)], tgt_v)
    acc = jnp.zeros((_L,), jnp.float32)
    for k in range(_NCH):
        is_ign = tgt_v[pl.ds(k * _L, _L)] == IGN_COL
        acc = acc + jnp.where(is_ign, S * LS, 0.0)
    row_v[...] = acc
    pltpu.sync_copy(row_v, out_hbm.at[wid])


def kernel(output, target, one_hot):
    wsum = _weighted_sum(output.T, target)[0, 0]
    parts = _sc_ign_term(target)
    return jnp.float32(CONST) + wsum + jnp.sum(parts)


# speedup vs baseline: 4.0947x; 1.1191x over previous
import functools

import jax
import jax.numpy as jnp
import numpy as np
from jax import lax
from jax.experimental import pallas as pl
from jax.experimental.pallas import tpu as pltpu
from jax.experimental.pallas import tpu_sc as plsc

V = 100000
B = 1024
IGN_COL = V - 100
S = float(np.float32(0.1 / (V - 2)))
C = 0.9
LS = float(np.log(np.float32(S)))
LC = float(np.log(np.float32(C)))
CONST = B * ((V - 2) * S * LS + C * LC)


_VB = 5000


def _wsum_body(t_ref, x_ref, acc_ref):
    @pl.when(pl.program_id(0) == 0)
    def _():
        acc_ref[0, 0] = 0.0

    x = x_ref[...]
    t = t_ref[...]
    v = lax.broadcasted_iota(jnp.int32, (_VB, B), 0) + pl.program_id(0) * _VB
    w = jnp.where(v == t, S - C, 0.0) - S
    w = jnp.where((v == IGN_COL) & (t != IGN_COL), w + S, w)
    acc_ref[0, 0] += jnp.sum(x * w)


def _weighted_sum(xt, target):
    return pl.pallas_call(
        _wsum_body,
        grid=(V // _VB,),
        in_specs=[
            pl.BlockSpec((1, B), lambda i: (0, 0)),
            pl.BlockSpec((_VB, B), lambda i: (i, 0)),
        ],
        out_specs=pl.BlockSpec(memory_space=pltpu.SMEM),
        out_shape=jax.ShapeDtypeStruct((1, 1), jnp.float32),
    )(target.reshape(1, B), xt)


_INFO = plsc.get_sparse_core_info()
_NC, _NS, _L = _INFO.num_cores, _INFO.num_subcores, _INFO.num_lanes
_NW = _NC * _NS
_BPW = B // _NW
_NCH = _BPW // _L

_MESH = plsc.VectorSubcoreMesh(core_axis_name="c", subcore_axis_name="s")


@functools.partial(
    pl.kernel,
    mesh=_MESH,
    out_type=jax.ShapeDtypeStruct((_NW, _L), jnp.float32),
    scratch_types=[
        pltpu.VMEM((_BPW,), jnp.int32),
        pltpu.VMEM((_L,), jnp.float32),
    ],
)
def _sc_ign_term(tgt_hbm, out_hbm, tgt_v, row_v):
    wid = lax.axis_index("s") * _NC + lax.axis_index("c")
    base = wid * _BPW
    pltpu.sync_copy(tgt_hbm.at[pl.ds(base, _BPW)], tgt_v)
    acc = jnp.zeros((_L,), jnp.float32)
    for k in range(_NCH):
        is_ign = tgt_v[pl.ds(k * _L, _L)] == IGN_COL
        acc = acc + jnp.where(is_ign, S * LS, 0.0)
    row_v[...] = acc
    pltpu.sync_copy(row_v, out_hbm.at[wid])


def kernel(output, target, one_hot):
    wsum = _weighted_sum(output.T, target)[0, 0]
    ign = jnp.float32(S * LS) * jnp.sum(jnp.where(target == IGN_COL, 1.0, 0.0))
    return jnp.float32(CONST) + wsum + ign
